# baseline (speedup 1.0000x reference)
import jax
import jax.numpy as jnp
from jax import lax
from jax.experimental import pallas as pl
from jax.experimental.pallas import tpu as pltpu
from jax.experimental.pallas import tpu_sc as plsc

_MEM_ROWS = 65536
_ASSETS = 512
_NUM_WORKERS = 32
_WROWS = _MEM_ROWS // _NUM_WORKERS
_R = 32
_N = _WROWS // _R
_NBUF = 4
_LA = 2
_NGRP = _N // _NBUF
_NLANE = _ASSETS // 16


def _shift_body(new_hbm, mem_hbm, out_hbm, buf, bbuf, newbuf,
                isem0, isem1, isem2, isem3,
                osem0, osem1, osem2, osem3,
                bsem, nsem):
    cid = lax.axis_index("c")
    sid = lax.axis_index("s")
    wid = sid * 2 + cid
    base = wid * _WROWS

    isems = (isem0, isem1, isem2, isem3)
    osems = (osem0, osem1, osem2, osem3)
    w_last = _NUM_WORKERS - 1

    new_cp = pltpu.make_async_copy(new_hbm, newbuf, nsem)
    bb_cp = pltpu.make_async_copy(
        mem_hbm.at[pl.ds(jnp.minimum(base + _WROWS, _MEM_ROWS - 8), 8)],
        bbuf, bsem)

    @pl.when(wid == w_last)
    def _stage_new():
        new_cp.start()

    @pl.when(wid < w_last)
    def _stage_boundary():
        bb_cp.start()

    def start_g(c, i):
        o = base + c * _R
        pltpu.make_async_copy(
            mem_hbm.at[pl.ds(o, _R)], buf.at[i], isems[i]).start()

    def wait_g(i):
        pltpu.make_async_copy(
            mem_hbm.at[pl.ds(0, _R)], buf.at[i], isems[i]).wait()

    def start_s(c, i):
        o = base + c * _R
        pltpu.make_async_copy(
            buf.at[i], out_hbm.at[pl.ds(o, _R)], osems[i]).start()

    def wait_s(i):
        pltpu.make_async_copy(
            buf.at[i], out_hbm.at[pl.ds(0, _R)], osems[i]).wait()

    def rotate(i):
        def group(g, carry):
            r0 = 8 * g
            for r in range(8):
                for cs in range(_NLANE):
                    col = cs * 16
                    buf[i, r0 + r, pl.ds(col, 16)] = (
                        buf[i, r0 + r + 1, pl.ds(col, 16)])
            return carry
        lax.fori_loop(0, _R // 8 - 1, group, 0)
        for r in range(24, _R - 1):
            for cs in range(_NLANE):
                col = cs * 16
                buf[i, r, pl.ds(col, 16)] = buf[i, r + 1, pl.ds(col, 16)]

    def chunk(c, jj):
        nb = (jj + 1) % _NBUF
        kb = (jj + _LA) % _NBUF

        @pl.when(c + _LA < _N)
        def _lookahead():
            @pl.when(c >= _NBUF - _LA)
            def _():
                wait_s(kb)
            start_g(c + _LA, kb)

        @pl.when(c < _N - 1)
        def _():
            wait_g(nb)

        rotate(jj)

        @pl.when(c < _N - 1)
        def _row31_next():
            for cs in range(_NLANE):
                col = cs * 16
                buf[jj, _R - 1, pl.ds(col, 16)] = buf[nb, 0, pl.ds(col, 16)]

        @pl.when(c == _N - 1)
        def _row31_tail():
            @pl.when(wid < w_last)
            def _():
                bb_cp.wait()
                for cs in range(_NLANE):
                    col = cs * 16
                    buf[jj, _R - 1, pl.ds(col, 16)] = bbuf[0, pl.ds(col, 16)]

            @pl.when(wid == w_last)
            def _():
                new_cp.wait()
                for cs in range(_NLANE):
                    col = cs * 16
                    buf[jj, _R - 1, pl.ds(col, 16)] = newbuf[pl.ds(col, 16)]

        start_s(c, jj)

    def group_body(g, carry):
        c0 = g * _NBUF
        for jj in range(_NBUF):
            chunk(c0 + jj, jj)
        return carry

    for c in range(_LA):
        start_g(c, c)
    wait_g(0)
    lax.fori_loop(0, _NGRP, group_body, 0)

    for c in range(_N - _NBUF, _N):
        wait_s(c % _NBUF)


@jax.jit
def _shift(new, memory):
    mesh = plsc.VectorSubcoreMesh(core_axis_name="c", subcore_axis_name="s")
    return pl.kernel(
        _shift_body,
        out_type=jax.ShapeDtypeStruct((_MEM_ROWS, _ASSETS), jnp.float32),
        mesh=mesh,
        scratch_types=(
            [pltpu.VMEM((_NBUF, _R, _ASSETS), jnp.float32),
             pltpu.VMEM((8, _ASSETS), jnp.float32),
             pltpu.VMEM((_ASSETS,), jnp.float32)]
            + [pltpu.SemaphoreType.DMA] * (2 * _NBUF + 2)
        ),
    )(new, memory)


def kernel(new, memory):
    return _shift(new, memory)


# speedup vs baseline: 1.1036x; 1.1036x over previous
import jax
import jax.numpy as jnp
from jax import lax
from jax.experimental import pallas as pl
from jax.experimental.pallas import tpu as pltpu
from jax.experimental.pallas import tpu_sc as plsc

_MEM_ROWS = 65536
_ASSETS = 512
_NUM_WORKERS = 32
_WROWS = _MEM_ROWS // _NUM_WORKERS
_R = 32
_N = _WROWS // _R
_NBUF = 6
_LA = 4
_NLANE = _ASSETS // 16


def _shift_body(new_hbm, mem_hbm, out_hbm, buf, bbuf, newbuf,
                isem0, isem1, isem2, isem3, isem4, isem5,
                osem0, osem1, osem2, osem3, osem4, osem5,
                bsem, nsem):
    cid = lax.axis_index("c")
    sid = lax.axis_index("s")
    wid = sid * 2 + cid
    base = wid * _WROWS

    isems = (isem0, isem1, isem2, isem3, isem4, isem5)
    osems = (osem0, osem1, osem2, osem3, osem4, osem5)
    w_last = _NUM_WORKERS - 1

    new_cp = pltpu.make_async_copy(new_hbm, newbuf, nsem)
    bb_cp = pltpu.make_async_copy(
        mem_hbm.at[pl.ds(jnp.minimum(base + _WROWS, _MEM_ROWS - 8), 8)],
        bbuf, bsem)

    @pl.when(wid == w_last)
    def _stage_new():
        new_cp.start()

    @pl.when(wid < w_last)
    def _stage_boundary():
        bb_cp.start()

    def for_buf(b, fn):
        for i in range(_NBUF):
            @pl.when(b == i)
            def _(i=i):
                fn(i)

    def start_g(c, b):
        o = base + c * _R
        for_buf(b, lambda i: pltpu.make_async_copy(
            mem_hbm.at[pl.ds(o, _R)], buf.at[i], isems[i]).start())

    def wait_g(b):
        for_buf(b, lambda i: pltpu.make_async_copy(
            mem_hbm.at[pl.ds(0, _R)], buf.at[i], isems[i]).wait())

    def start_s(c, b):
        o = base + c * _R
        for_buf(b, lambda i: pltpu.make_async_copy(
            buf.at[i], out_hbm.at[pl.ds(o, _R)], osems[i]).start())

    def wait_s(b):
        for_buf(b, lambda i: pltpu.make_async_copy(
            buf.at[i], out_hbm.at[pl.ds(0, _R)], osems[i]).wait())

    def rotate(b):
        def group(g, carry):
            r0 = 8 * g
            for r in range(8):
                for cs in range(_NLANE):
                    col = cs * 16
                    buf[b, r0 + r, pl.ds(col, 16)] = (
                        buf[b, r0 + r + 1, pl.ds(col, 16)])
            return carry
        lax.fori_loop(0, _R // 8 - 1, group, 0)
        for r in range(24, _R - 1):
            for cs in range(_NLANE):
                col = cs * 16
                buf[b, r, pl.ds(col, 16)] = buf[b, r + 1, pl.ds(col, 16)]

    def chunk_body(c, carry):
        b = lax.rem(c, _NBUF)
        b1 = lax.rem(c + 1, _NBUF)
        kb = lax.rem(c + _LA, _NBUF)

        @pl.when(c + _LA < _N)
        def _lookahead():
            @pl.when(c >= _NBUF - _LA)
            def _():
                wait_s(kb)
            start_g(c + _LA, kb)

        @pl.when(c < _N - 1)
        def _():
            wait_g(b1)

        rotate(b)

        @pl.when(c < _N - 1)
        def _row31_next():
            def mv(i):
                j = (i + 1) % _NBUF
                for cs in range(_NLANE):
                    col = cs * 16
                    buf[i, _R - 1, pl.ds(col, 16)] = buf[j, 0, pl.ds(col, 16)]
            for_buf(b, mv)

        @pl.when(c == _N - 1)
        def _row31_tail():
            @pl.when(wid < w_last)
            def _():
                bb_cp.wait()
                for_buf(b, lambda i: _copy_row(i, bbuf))

            @pl.when(wid == w_last)
            def _():
                new_cp.wait()
                for_buf(b, lambda i: _copy_row_flat(i, newbuf))

        start_s(c, b)
        return carry

    def _copy_row(i, src):
        for cs in range(_NLANE):
            col = cs * 16
            buf[i, _R - 1, pl.ds(col, 16)] = src[0, pl.ds(col, 16)]

    def _copy_row_flat(i, src):
        for cs in range(_NLANE):
            col = cs * 16
            buf[i, _R - 1, pl.ds(col, 16)] = src[pl.ds(col, 16)]

    for c in range(_LA):
        start_g(c, c)
    wait_g(0)
    lax.fori_loop(0, _N, chunk_body, 0)

    for c in range(_N - _NBUF, _N):
        i = c % _NBUF
        pltpu.make_async_copy(
            buf.at[i], out_hbm.at[pl.ds(0, _R)], osems[i]).wait()


@jax.jit
def _shift(new, memory):
    mesh = plsc.VectorSubcoreMesh(core_axis_name="c", subcore_axis_name="s")
    return pl.kernel(
        _shift_body,
        out_type=jax.ShapeDtypeStruct((_MEM_ROWS, _ASSETS), jnp.float32),
        mesh=mesh,
        scratch_types=(
            [pltpu.VMEM((_NBUF, _R, _ASSETS), jnp.float32),
             pltpu.VMEM((8, _ASSETS), jnp.float32),
             pltpu.VMEM((_ASSETS,), jnp.float32)]
            + [pltpu.SemaphoreType.DMA] * (2 * _NBUF + 2)
        ),
    )(new, memory)


def kernel(new, memory):
    return _shift(new, memory)
